# baseline (speedup 1.0000x reference)
import functools

import jax
import jax.numpy as jnp
from jax import lax
from jax.experimental import pallas as pl
from jax.experimental.pallas import tpu as pltpu
from jax.experimental.pallas import tpu_sc as plsc

_BATCH = 16384
_EMB = 32
_HID = 64
_NC = 2
_NS = 16
_NW = _NC * _NS
_BPW = _BATCH // _NW
_CHUNK = 128
_NK = _BPW // _CHUNK


def _sc_gather_body(utab, itab, uidx, iidx, u_out, i_out,
                    uidx_v, iidx_v, urows_v, irows_v, sem):
    wid = lax.axis_index("s") * _NC + lax.axis_index("c")
    base = wid * _BPW
    row0 = wid * _NK
    pltpu.sync_copy(uidx.at[pl.ds(row0, _NK)], uidx_v)
    pltpu.sync_copy(iidx.at[pl.ds(row0, _NK)], iidx_v)
    copies = []
    for k in range(_NK):
        copies.append(pltpu.async_copy(
            utab.at[uidx_v.at[k]], urows_v.at[pl.ds(k * _CHUNK, _CHUNK)], sem))
        copies.append(pltpu.async_copy(
            itab.at[iidx_v.at[k]], irows_v.at[pl.ds(k * _CHUNK, _CHUNK)], sem))
    for c in copies:
        c.wait()
    pltpu.sync_copy(urows_v, u_out.at[pl.ds(base, _BPW)])
    pltpu.sync_copy(irows_v, i_out.at[pl.ds(base, _BPW)])


_sc_gather = functools.partial(
    pl.kernel,
    mesh=plsc.VectorSubcoreMesh(core_axis_name="c", subcore_axis_name="s"),
    out_type=[
        jax.ShapeDtypeStruct((_BATCH, _EMB), jnp.float32),
        jax.ShapeDtypeStruct((_BATCH, _EMB), jnp.float32),
    ],
    scratch_types=[
        pltpu.VMEM((_NK, _CHUNK), jnp.int32),
        pltpu.VMEM((_NK, _CHUNK), jnp.int32),
        pltpu.VMEM((_BPW, _EMB), jnp.float32),
        pltpu.VMEM((_BPW, _EMB), jnp.float32),
        pltpu.SemaphoreType.DMA,
    ],
    compiler_params=pltpu.CompilerParams(use_tc_tiling_on_sc=False),
)(_sc_gather_body)


_BN = 2048


def _mlp_body(u_ref, i_ref, w1u_ref, w1i_ref, b1_ref, w2_ref, b2_ref, o_ref):
    h = jnp.dot(u_ref[...], w1u_ref[...], preferred_element_type=jnp.float32)
    h = h + jnp.dot(i_ref[...], w1i_ref[...], preferred_element_type=jnp.float32)
    h = jnp.maximum(h + b1_ref[...], 0.0)
    o_ref[...] = jnp.dot(h, w2_ref[...], preferred_element_type=jnp.float32) + b2_ref[...]


_mlp = pl.pallas_call(
    _mlp_body,
    grid=(_BATCH // _BN,),
    in_specs=[
        pl.BlockSpec((_BN, _EMB), lambda n: (n, 0)),
        pl.BlockSpec((_BN, _EMB), lambda n: (n, 0)),
        pl.BlockSpec((_EMB, _HID), lambda n: (0, 0)),
        pl.BlockSpec((_EMB, _HID), lambda n: (0, 0)),
        pl.BlockSpec((1, _HID), lambda n: (0, 0)),
        pl.BlockSpec((_HID, 1), lambda n: (0, 0)),
        pl.BlockSpec((1, 1), lambda n: (0, 0)),
    ],
    out_specs=pl.BlockSpec((_BN, 1), lambda n: (n, 0)),
    out_shape=jax.ShapeDtypeStruct((_BATCH, 1), jnp.float32),
)


def kernel(users, items, user_table, item_table, W1, b1, W2, b2):
    uidx = users.reshape(_NW * _NK, _CHUNK)
    iidx = items.reshape(_NW * _NK, _CHUNK)
    urows, irows = _sc_gather(user_table, item_table, uidx, iidx)
    w1u = W1[:, :_EMB].T
    w1i = W1[:, _EMB:].T
    out = _mlp(urows, irows, w1u, w1i, b1.reshape(1, _HID),
               W2.reshape(1, _HID).T, b2.reshape(1, 1))
    return out.reshape(_BATCH)
